# baseline (device time: 183534 ns/iter reference)
import jax
import jax.numpy as jnp
from jax import lax
from jax.experimental import pallas as pl
from jax.experimental.pallas import tpu as pltpu

N_DEV = 4
N_SUB = 12


def kernel(x):
    m, n = x.shape
    ch = m // N_DEV
    ch2 = ch // 2
    n2 = n // 2

    def body(x_ref, out_ref, recv_ref, send_sems, recv_sems, credit_sems):
        my = lax.axis_index("i")
        left = jnp.mod(my - 1, N_DEV)
        right = jnp.mod(my + 1, N_DEV)
        send_to = (right, left)
        recv_from = (left, right)

        barrier_sem = pltpu.get_barrier_semaphore()
        for nbr in (left, right):
            pl.semaphore_signal(
                barrier_sem, inc=1,
                device_id=(nbr,), device_id_type=pl.DeviceIdType.MESH,
            )
        pl.semaphore_wait(barrier_sem, 2)

        out_ref[...] = x_ref[...]

        def cols(d):
            return pl.ds(d * n2, n2)

        def chunk_send(d, h):
            return jnp.mod(my - h, N_DEV) if d == 0 else jnp.mod(my + h, N_DEV)

        def chunk_acc(d, h):
            return (
                jnp.mod(my - h - 1, N_DEV) if d == 0
                else jnp.mod(my + h + 1, N_DEV)
            )

        rc = (jnp.mod(my + 1, N_DEV), jnp.mod(my - 1, N_DEV))

        def make_rdma(d, t):
            j = t % 2
            if t <= 5:
                row0 = chunk_send(d, t // 2) * ch + j * ch2
                src = out_ref.at[pl.ds(row0, ch2), cols(d)]
            elif t <= 7:
                src = out_ref.at[pl.ds(rc[d] * ch + j * ch2, ch2), cols(d)]
            else:
                src = recv_ref.at[d, (t - 2) % 4]
            slot = t % 4
            return pltpu.make_async_remote_copy(
                src_ref=src,
                dst_ref=recv_ref.at[d, slot],
                send_sem=send_sems.at[d, slot],
                recv_sem=recv_sems.at[d, slot],
                device_id=(send_to[d],),
                device_id_type=pl.DeviceIdType.MESH,
            )

        rdmas = [[None] * N_SUB, [None] * N_SUB]
        for t in (0, 1):
            for d in range(2):
                rdmas[d][t] = make_rdma(d, t)
                rdmas[d][t].start()

        for t in range(2, N_SUB + 2):
            r = t - 2
            j = r % 2
            for d in range(2):
                rdmas[d][r].wait_recv()
                if r <= 5:
                    row0 = chunk_acc(d, r // 2) * ch + j * ch2
                    out_ref[pl.ds(row0, ch2), cols(d)] = (
                        out_ref[pl.ds(row0, ch2), cols(d)]
                        + recv_ref[d, r % 4]
                    )
                    pl.semaphore_signal(
                        credit_sems.at[d, r % 4], inc=1,
                        device_id=(recv_from[d],),
                        device_id_type=pl.DeviceIdType.MESH,
                    )
                else:
                    g = (r - 6) // 2
                    c = (
                        jnp.mod(my - g, N_DEV) if d == 0
                        else jnp.mod(my + g, N_DEV)
                    )
                    out_ref[pl.ds(c * ch + j * ch2, ch2), cols(d)] = (
                        recv_ref[d, r % 4]
                    )
            if t <= N_SUB - 1:
                for d in range(2):
                    if t >= 4:
                        rdmas[d][t - 4].wait_send()
                if t in (10, 11):
                    for d in range(2):
                        rdmas[d][t - 2].wait_send()
                        pl.semaphore_signal(
                            credit_sems.at[d, t % 4], inc=1,
                            device_id=(recv_from[d],),
                            device_id_type=pl.DeviceIdType.MESH,
                        )
                for d in range(2):
                    if t >= 4:
                        pl.semaphore_wait(credit_sems.at[d, t % 4], 1)
                    rdmas[d][t] = make_rdma(d, t)
                    rdmas[d][t].start()
            else:
                for d in range(2):
                    rdmas[d][t - 2].wait_send()

    out_shape = jax.ShapeDtypeStruct((m, n), jnp.bfloat16)
    return pl.pallas_call(
        body,
        out_shape=out_shape,
        in_specs=[pl.BlockSpec(memory_space=pltpu.VMEM)],
        out_specs=pl.BlockSpec(memory_space=pltpu.VMEM),
        scratch_shapes=[
            pltpu.VMEM((2, 4, ch2, n2), jnp.bfloat16),
            pltpu.SemaphoreType.DMA((2, 4)),
            pltpu.SemaphoreType.DMA((2, 4)),
            pltpu.SemaphoreType.REGULAR((2, 4)),
        ],
        compiler_params=pltpu.CompilerParams(
            collective_id=0,
            vmem_limit_bytes=100 * 1024 * 1024,
        ),
    )(x.astype(jnp.bfloat16))


# device time: 161596 ns/iter; 1.1358x vs baseline; 1.1358x over previous
import jax
import jax.numpy as jnp
from jax import lax
from jax.experimental import pallas as pl
from jax.experimental.pallas import tpu as pltpu

N_DEV = 4
N_SUB = 12


def kernel(x):
    m, n = x.shape
    ch = m // N_DEV
    ch2 = ch // 2
    n2 = n // 2

    def body(x_ref, out_ref, w_ref, recv_ref, stage_ref,
             send_sems, recv_sems, credit_sems, copy_sems):
        my = lax.axis_index("i")
        left = jnp.mod(my - 1, N_DEV)
        right = jnp.mod(my + 1, N_DEV)
        send_to = (right, left)
        recv_from = (left, right)

        barrier_sem = pltpu.get_barrier_semaphore()
        for nbr in (left, right):
            pl.semaphore_signal(
                barrier_sem, inc=1,
                device_id=(nbr,), device_id_type=pl.DeviceIdType.MESH,
            )
        pl.semaphore_wait(barrier_sem, 2)

        CAST_ORDER = (0, 3, 1, 2)

        def start_stage(k, slot):
            c = jnp.mod(my + CAST_ORDER[k], N_DEV)
            cp = pltpu.make_async_copy(
                x_ref.at[pl.ds(c * ch, ch), :],
                stage_ref.at[slot],
                copy_sems.at[slot],
            )
            cp.start()
            return cp

        def cast_stage(k, slot):
            c = jnp.mod(my + CAST_ORDER[k], N_DEV)
            w_ref[0, pl.ds(c * ch, ch), :] = (
                stage_ref[slot, :, :n2].astype(jnp.bfloat16)
            )
            w_ref[1, pl.ds(c * ch, ch), :] = (
                stage_ref[slot, :, n2:].astype(jnp.bfloat16)
            )

        def start_stage_half(half):
            row0 = my * ch + half * ch2
            cp = pltpu.make_async_copy(
                x_ref.at[pl.ds(row0, ch2), :],
                stage_ref.at[0, pl.ds(half * ch2, ch2), :],
                copy_sems.at[half],
            )
            cp.start()
            return cp

        def cast_stage_half(half):
            row0 = my * ch + half * ch2
            sl = stage_ref[0, pl.ds(half * ch2, ch2), :]
            w_ref[0, pl.ds(row0, ch2), :] = sl[:, :n2].astype(jnp.bfloat16)
            w_ref[1, pl.ds(row0, ch2), :] = sl[:, n2:].astype(jnp.bfloat16)

        cph = [start_stage_half(0), start_stage_half(1)]
        cph[0].wait()
        cast_stage_half(0)

        def chunk_send(d, h):
            return jnp.mod(my - h, N_DEV) if d == 0 else jnp.mod(my + h, N_DEV)

        def chunk_acc(d, h):
            return (
                jnp.mod(my - h - 1, N_DEV) if d == 0
                else jnp.mod(my + h + 1, N_DEV)
            )

        rc = (jnp.mod(my + 1, N_DEV), jnp.mod(my - 1, N_DEV))

        def make_rdma(d, t):
            j = t % 2
            if t <= 5:
                row0 = chunk_send(d, t // 2) * ch + j * ch2
                src = w_ref.at[d, pl.ds(row0, ch2), :]
            elif t <= 7:
                src = w_ref.at[d, pl.ds(rc[d] * ch + j * ch2, ch2), :]
            else:
                src = recv_ref.at[d, (t - 2) % 4]
            slot = t % 4
            return pltpu.make_async_remote_copy(
                src_ref=src,
                dst_ref=recv_ref.at[d, slot],
                send_sem=send_sems.at[d, slot],
                recv_sem=recv_sems.at[d, slot],
                device_id=(send_to[d],),
                device_id_type=pl.DeviceIdType.MESH,
            )

        rdmas = [[None] * N_SUB, [None] * N_SUB]

        for d in range(2):
            rdmas[d][0] = make_rdma(d, 0)
            rdmas[d][0].start()
        cph[1].wait()
        cast_stage_half(1)
        for d in range(2):
            rdmas[d][1] = make_rdma(d, 1)
            rdmas[d][1].start()
        cps = [None, None]
        cps[1] = start_stage(1, 1)
        cps[0] = start_stage(2, 0)
        cps[1].wait()
        cast_stage(1, 1)
        cps[1] = start_stage(3, 1)
        cps[0].wait()
        cast_stage(2, 0)
        cps[1].wait()
        cast_stage(3, 1)

        for t in range(2, N_SUB + 2):
            r = t - 2
            j = r % 2
            for d in range(2):
                rdmas[d][r].wait_recv()
                if r <= 5:
                    a = chunk_acc(d, r // 2)
                    w_ref[d, pl.ds(a * ch + j * ch2, ch2), :] = (
                        w_ref[d, pl.ds(a * ch + j * ch2, ch2), :]
                        + recv_ref[d, r % 4]
                    )
                    pl.semaphore_signal(
                        credit_sems.at[d, r % 4], inc=1,
                        device_id=(recv_from[d],),
                        device_id_type=pl.DeviceIdType.MESH,
                    )
                else:
                    g = (r - 6) // 2
                    c = (
                        jnp.mod(my - g, N_DEV) if d == 0
                        else jnp.mod(my + g, N_DEV)
                    )
                    if d == 0:
                        out_ref[pl.ds(c * ch + j * ch2, ch2), :n2] = (
                            recv_ref[0, r % 4]
                        )
                    else:
                        out_ref[pl.ds(c * ch + j * ch2, ch2), n2:] = (
                            recv_ref[1, r % 4]
                        )
            if t in (6, 7):
                sj = t % 2
                out_ref[pl.ds(rc[0] * ch + sj * ch2, ch2), :n2] = (
                    w_ref[0, pl.ds(rc[0] * ch + sj * ch2, ch2), :]
                )
                out_ref[pl.ds(rc[1] * ch + sj * ch2, ch2), n2:] = (
                    w_ref[1, pl.ds(rc[1] * ch + sj * ch2, ch2), :]
                )
            if t <= N_SUB - 1:
                for d in range(2):
                    if t >= 4:
                        rdmas[d][t - 4].wait_send()
                if t in (10, 11):
                    for d in range(2):
                        rdmas[d][t - 2].wait_send()
                        pl.semaphore_signal(
                            credit_sems.at[d, t % 4], inc=1,
                            device_id=(recv_from[d],),
                            device_id_type=pl.DeviceIdType.MESH,
                        )
                for d in range(2):
                    if t >= 4:
                        pl.semaphore_wait(credit_sems.at[d, t % 4], 1)
                    rdmas[d][t] = make_rdma(d, t)
                    rdmas[d][t].start()
            else:
                for d in range(2):
                    rdmas[d][t - 2].wait_send()

    out_shape = jax.ShapeDtypeStruct((m, n), jnp.bfloat16)
    return pl.pallas_call(
        body,
        out_shape=out_shape,
        in_specs=[pl.BlockSpec(memory_space=pl.ANY)],
        out_specs=pl.BlockSpec(memory_space=pltpu.VMEM),
        scratch_shapes=[
            pltpu.VMEM((2, m, n2), jnp.bfloat16),
            pltpu.VMEM((2, 4, ch2, n2), jnp.bfloat16),
            pltpu.VMEM((2, ch, n), jnp.float32),
            pltpu.SemaphoreType.DMA((2, 4)),
            pltpu.SemaphoreType.DMA((2, 4)),
            pltpu.SemaphoreType.REGULAR((2, 4)),
            pltpu.SemaphoreType.DMA((2,)),
        ],
        compiler_params=pltpu.CompilerParams(
            collective_id=0,
            vmem_limit_bytes=100 * 1024 * 1024,
        ),
    )(x)
